# Initial kernel scaffold; baseline (speedup 1.0000x reference)
import jax
import jax.numpy as jnp
from jax.experimental import pallas as pl


def kernel(x, dfwd, dinv, lift_wx, lift_wg, lift_b, q_w1t, q_b1, q_w2t, q_b2, l0_amix, l0_bmix, l0_w1t, l0_b1, l0_wpack, l0_bpack, l1_amix, l1_bmix, l1_w1t, l1_b1, l1_wpack, l1_bpack, l2_amix, l2_bmix, l2_w1t, l2_b1, l2_wpack, l2_bpack, l3_amix, l3_bmix, l3_w1t, l3_b1, l3_wpack, l3_bpack):
    raise NotImplementedError("write your pallas kernel here")


# speedup vs baseline: 1.4840x; 1.4840x over previous
import functools
import math

import jax
import jax.numpy as jnp
from jax import lax
from jax.experimental import pallas as pl
from jax.experimental.pallas import tpu as pltpu

_G = 4


def _gelu(x):
    return 0.5 * x * (1.0 + lax.erf(x * jnp.float32(0.7071067811865476)))


def _fno_kernel(xp_ref, dfwd_ref, coef_ref, lwx_ref, lwg_ref, lb_ref,
                am_ref, bm_ref, w1_ref, b1_ref, wpk_ref, bpk_ref,
                qw1_ref, qb1_ref, qw2_ref, qb2_ref, o_ref, xs_ref,
                *, wp, s, mt, cw):
    n = dfwd_ref.shape[0]
    li = pl.program_id(1)
    iota = lax.broadcasted_iota(jnp.int32, (1, n), 1)
    col = iota % wp

    @pl.when(li == 0)
    def _lift():
        hh = iota // wp
        inside = (hh < s) & (col < s)
        inv = jnp.float32(1.0 / (s - 1))
        gx = hh.astype(jnp.float32) * inv
        gy = col.astype(jnp.float32) * inv
        pos = lwg_ref[:, 0:1] * gx + lwg_ref[:, 1:2] * gy + lb_ref[...]
        for g in range(_G):
            xg = xp_ref[0, g:g + 1, :]
            row = lwx_ref[:, 0:1] * xg + pos
            xs_ref[g * cw:(g + 1) * cw, :] = jnp.where(inside, row, 0.0)

    x = xs_ref[...]
    x2 = jnp.dot(x, dfwd_ref[...], preferred_element_type=jnp.float32)

    coef = coef_ref[...]
    out2_rows = []
    for g in range(_G):
        x2g = x2[g * cw:(g + 1) * cw, :]
        rot = jnp.concatenate([x2g[:, mt:], x2g[:, :mt]], axis=1)
        acc = x2g[0:1, :] * am_ref[0, 0] + rot[0:1, :] * bm_ref[0, 0]
        for ci in range(1, cw):
            acc = (acc + x2g[ci:ci + 1, :] * am_ref[0, ci]
                   + rot[ci:ci + 1, :] * bm_ref[0, ci])
        out2_rows.append(acc * coef)
    out2 = jnp.concatenate(out2_rows, axis=0)

    x1 = lax.dot_general(out2, dfwd_ref[...], (((1,), (1,)), ((), ())),
                         preferred_element_type=jnp.float32)

    h = _gelu(jnp.dot(w1_ref[0], x1, preferred_element_type=jnp.float32)
              + b1_ref[0])

    def shifted(k):
        k = k % n
        if k == 0:
            return x
        return jnp.concatenate([x[:, k:], x[:, :k]], axis=1)

    xo = jnp.dot(wpk_ref[0, 0], h, preferred_element_type=jnp.float32)
    t = 1
    for dh in (-1, 0, 1):
        for dw in (-1, 0, 1):
            main = shifted(dh * wp + dw)
            if dw != 0:
                fix = shifted(dh * wp + dw - dw * wp)
                edge = (col == (wp - 1)) if dw == 1 else (col == 0)
                main = jnp.where(edge, fix, main)
            xo = xo + jnp.dot(wpk_ref[0, t], main,
                              preferred_element_type=jnp.float32)
            t += 1

    y = x + xo + bpk_ref[0]
    y = jnp.where(li < 3, _gelu(y), y)
    xs_ref[...] = y

    @pl.when(li == 3)
    def _proj():
        yf = xs_ref[...]
        hq = _gelu(jnp.dot(qw1_ref[...], yf,
                           preferred_element_type=jnp.float32) + qb1_ref[...])
        o_ref[0] = (jnp.dot(qw2_ref[...], hq,
                            preferred_element_type=jnp.float32) + qb2_ref[...])


def _bd(w):
    return jnp.kron(jnp.eye(_G, dtype=w.dtype), w)


def kernel(x, dfwd, dinv, lift_wx, lift_wg, lift_b, q_w1t, q_b1, q_w2t, q_b2,
           l0_amix, l0_bmix, l0_w1t, l0_b1, l0_wpack, l0_bpack,
           l1_amix, l1_bmix, l1_w1t, l1_b1, l1_wpack, l1_bpack,
           l2_amix, l2_bmix, l2_w1t, l2_b1, l2_wpack, l2_bpack,
           l3_amix, l3_bmix, l3_w1t, l3_b1, l3_wpack, l3_bpack):
    B, S, _, _ = x.shape
    n, m2 = dfwd.shape
    mt = m2 // 2
    wp = int(round(math.sqrt(n)))
    pad = wp - S
    width = lift_wx.shape[0]
    ncore = B // _G
    R = _G * width

    xp = jnp.pad(x[..., 0], ((0, 0), (0, pad), (0, pad))).reshape(ncore, _G, n)

    coef = dinv[:mt, 0]
    coef2 = jnp.concatenate([coef, coef]).reshape(1, m2)

    amix = jnp.stack([l0_amix, l1_amix, l2_amix, l3_amix])
    bmix = jnp.stack([l0_bmix, l1_bmix, l2_bmix, l3_bmix])
    w1bd = jnp.stack([_bd(w) for w in (l0_w1t, l1_w1t, l2_w1t, l3_w1t)])
    b1bd = jnp.stack([jnp.tile(b, (_G, 1))
                      for b in (l0_b1, l1_b1, l2_b1, l3_b1)])
    wpkbd = jnp.stack([
        jnp.stack([_bd(wpk[:, t * width:(t + 1) * width]) for t in range(10)])
        for wpk in (l0_wpack, l1_wpack, l2_wpack, l3_wpack)])
    bpkbd = jnp.stack([jnp.tile(b, (_G, 1))
                       for b in (l0_bpack, l1_bpack, l2_bpack, l3_bpack)])
    qw1bd = _bd(q_w1t)
    qb1bd = jnp.tile(q_b1, (_G, 1))
    qw2bd = _bd(q_w2t)
    qb2bd = jnp.tile(q_b2, (_G, 1))

    hid = q_w1t.shape[0]
    odim = q_w2t.shape[0]
    kern = functools.partial(_fno_kernel, wp=wp, s=S, mt=mt, cw=width)
    const = lambda i, j: (0, 0)
    per_layer3 = lambda i, j: (j, 0, 0)
    per_layer4 = lambda i, j: (j, 0, 0, 0)
    per_core = lambda i, j: (i, 0, 0)

    out = pl.pallas_call(
        kern,
        out_shape=jax.ShapeDtypeStruct((ncore, _G * odim, n), jnp.float32),
        grid=(ncore, 4),
        in_specs=[
            pl.BlockSpec((1, _G, n), per_core),
            pl.BlockSpec((n, m2), const),
            pl.BlockSpec((1, m2), const),
            pl.BlockSpec((width, 1), const),
            pl.BlockSpec((width, 2), const),
            pl.BlockSpec((width, 1), const),
            pl.BlockSpec((1, width, width, m2), per_layer4),
            pl.BlockSpec((1, width, width, m2), per_layer4),
            pl.BlockSpec((1, R, R), per_layer3),
            pl.BlockSpec((1, R, 1), per_layer3),
            pl.BlockSpec((1, 10, R, R), per_layer4),
            pl.BlockSpec((1, R, 1), per_layer3),
            pl.BlockSpec((_G * hid, R), const),
            pl.BlockSpec((_G * hid, 1), const),
            pl.BlockSpec((_G * odim, _G * hid), const),
            pl.BlockSpec((_G * odim, 1), const),
        ],
        out_specs=pl.BlockSpec((1, _G * odim, n), per_core),
        scratch_shapes=[pltpu.VMEM((R, n), jnp.float32)],
        compiler_params=pltpu.CompilerParams(
            dimension_semantics=("parallel", "arbitrary")),
    )(xp, dfwd, coef2, lift_wx, lift_wg, lift_b,
      amix, bmix, w1bd, b1bd, wpkbd, bpkbd,
      qw1bd, qb1bd, qw2bd, qb2bd)

    out = out.reshape(B, odim, wp, wp)[:, :, :S, :S]
    return out.transpose(0, 2, 3, 1)
